# baseline (device time: 111385 ns/iter reference)
import jax
import jax.numpy as jnp
from jax import lax
from jax.experimental import pallas as pl
from jax.experimental.pallas import tpu as pltpu

N_DEV = 32
B = 128
D = 128
H = 256

GROUPS = [(24, 32), (16, 24), (8, 16), (4, 8), (2, 4), (1, 2)]
RS_GROUPS = [(1, 9), (9, 17), (17, 25), (25, 32)]


def kernel(x, Win0, Wout0, Win1, Wout1, Win2, Wout2):
    def body(x_ref, win0, wout0, win1, wout1, win2, wout2, out_ref,
             xg, pg, ps, send_sems, recv_sems):
        my = lax.axis_index("i")
        n = jnp.int32(N_DEV)

        xg[0:1] = x_ref[...].astype(jnp.bfloat16).reshape(1, B, D)

        def layer(win_ref, wout_ref):
            w_in = win_ref[...].astype(jnp.bfloat16)
            w_out = wout_ref[...].astype(jnp.bfloat16)

            def f_chunks(xs):
                m = xs.shape[0]
                h = jnp.dot(xs.reshape(m * B, D), w_in,
                            preferred_element_type=jnp.float32)
                h = jnp.maximum(h, 0.0).astype(jnp.bfloat16)
                p = jnp.dot(h, w_out, preferred_element_type=jnp.float32)
                return p.reshape(m, B, D)

            for o in range(1, N_DEV):
                pltpu.make_async_remote_copy(
                    src_ref=xg.at[0], dst_ref=xg.at[N_DEV - o],
                    send_sem=send_sems.at[0, o],
                    recv_sem=recv_sems.at[0, N_DEV - o],
                    device_id=(lax.rem(my + o, n),),
                    device_id_type=pl.DeviceIdType.MESH,
                ).start()

            pg[0:1] = f_chunks(xg[0:1]).astype(jnp.bfloat16)

            for lo, hi in GROUPS:
                for o in range(lo, hi):
                    pltpu.make_async_remote_copy(
                        src_ref=xg.at[0], dst_ref=xg.at[o],
                        send_sem=send_sems.at[0, o],
                        recv_sem=recv_sems.at[0, o],
                        device_id=(my,),
                        device_id_type=pl.DeviceIdType.MESH,
                    ).wait_recv()
                ps[lo:hi] = f_chunks(xg[lo:hi]).astype(jnp.bfloat16)
                for o in range(lo, hi):
                    pltpu.make_async_remote_copy(
                        src_ref=ps.at[o], dst_ref=pg.at[N_DEV - o],
                        send_sem=send_sems.at[1, o],
                        recv_sem=recv_sems.at[1, N_DEV - o],
                        device_id=(lax.rem(my + o, n),),
                        device_id_type=pl.DeviceIdType.MESH,
                    ).start()

            x_new = pg[0].astype(jnp.float32)
            for lo, hi in RS_GROUPS:
                for o in range(lo, hi):
                    pltpu.make_async_remote_copy(
                        src_ref=ps.at[0], dst_ref=pg.at[o],
                        send_sem=send_sems.at[1, o],
                        recv_sem=recv_sems.at[1, o],
                        device_id=(my,),
                        device_id_type=pl.DeviceIdType.MESH,
                    ).wait_recv()
                x_new = x_new + jnp.sum(pg[lo:hi].astype(jnp.float32), axis=0)

            for o in range(1, N_DEV):
                dst = (lax.rem(my + o, n),)
                pltpu.make_async_remote_copy(
                    src_ref=xg.at[0], dst_ref=xg.at[N_DEV - o],
                    send_sem=send_sems.at[0, o],
                    recv_sem=recv_sems.at[0, N_DEV - o],
                    device_id=dst, device_id_type=pl.DeviceIdType.MESH,
                ).wait_send()
                pltpu.make_async_remote_copy(
                    src_ref=ps.at[o], dst_ref=pg.at[N_DEV - o],
                    send_sem=send_sems.at[1, o],
                    recv_sem=recv_sems.at[1, N_DEV - o],
                    device_id=dst, device_id_type=pl.DeviceIdType.MESH,
                ).wait_send()

            xg[0:1] = x_new.astype(jnp.bfloat16).reshape(1, B, D)
            return x_new

        layer(win0, wout0)
        layer(win1, wout1)
        x_out = layer(win2, wout2)
        out_ref[...] = x_out

    return pl.pallas_call(
        body,
        out_shape=jax.ShapeDtypeStruct((B, D), jnp.float32),
        in_specs=[pl.BlockSpec(memory_space=pltpu.VMEM)] * 7,
        out_specs=pl.BlockSpec(memory_space=pltpu.VMEM),
        scratch_shapes=[
            pltpu.VMEM((N_DEV, B, D), jnp.bfloat16),
            pltpu.VMEM((N_DEV, B, D), jnp.bfloat16),
            pltpu.VMEM((N_DEV, B, D), jnp.bfloat16),
            pltpu.SemaphoreType.DMA((2, N_DEV)),
            pltpu.SemaphoreType.DMA((2, N_DEV)),
        ],
    )(x, Win0, Wout0, Win1, Wout1, Win2, Wout2)
